# Initial kernel scaffold; baseline (speedup 1.0000x reference)
import jax
import jax.numpy as jnp
from jax.experimental import pallas as pl


def kernel(x, edge_index, temp, W1, b1, W2, b2):
    raise NotImplementedError("write your pallas kernel here")


# speedup vs baseline: 38.7363x; 38.7363x over previous
from math import comb

import jax
import jax.numpy as jnp
from jax import lax
from jax.experimental import pallas as pl
from jax.experimental.pallas import tpu as pltpu
from jax.experimental.pallas import tpu_sc as plsc

N = 10000
E = 320000
K = 10
C = 64
CP = 128
NC = 2
NS = 16
NW = NC * NS
N_PAD = 10240
ROWS_W = N_PAD // NW
E_W = E // NW
CHUNK = 80
NCHUNK = E_W // CHUNK
RC = 80
NSUB = ROWS_W // RC
ROWS_SC = N_PAD // NS

f32 = jnp.float32
i32 = jnp.int32


def _mesh():
    return plsc.VectorSubcoreMesh(core_axis_name="c", subcore_axis_name="s")


def _params():
    return pltpu.CompilerParams(needs_layout_passes=False)


def _wid():
    return lax.axis_index("s") * NC + lax.axis_index("c")


def _rsqrt_pos(x):
    yi = jnp.full((16,), 0x5F3759DF, i32) - lax.shift_right_logical(
        plsc.bitcast(x, i32), 1)
    y = plsc.bitcast(yi, f32)
    for _ in range(4):
        y = y * (1.5 - 0.5 * x * y * y)
    return y


def _deg_body(rowr_hbm, degp_hbm, row_v, deg_v):
    wid = _wid()
    pltpu.sync_copy(rowr_hbm.at[wid], row_v)
    zero16 = jnp.zeros((16,), f32)

    @pl.loop(0, N_PAD // 16)
    def _(i):
        deg_v[pl.ds(i * 16, 16)] = zero16

    ones16 = jnp.ones((16,), f32)

    @pl.loop(0, NCHUNK)
    def _(i):
        for k in range(CHUNK // 16):
            idx = row_v[i, pl.ds(k * 16, 16)]
            plsc.addupdate_scatter(deg_v, (idx,), ones16)

    pltpu.sync_copy(deg_v, degp_hbm.at[pl.ds(wid * N_PAD, N_PAD)])


def _deg_partial(row_r):
    return pl.kernel(
        _deg_body,
        out_type=jax.ShapeDtypeStruct((NW * N_PAD,), f32),
        mesh=_mesh(),
        compiler_params=_params(),
        scratch_types=[
            pltpu.VMEM((NCHUNK, CHUNK), i32),
            pltpu.VMEM((N_PAD,), f32),
        ],
    )(row_r)


def _prep_body(degp_hbm, h_hbm, ck_hbm, disb_hbm, z_hbm, zt_hbm,
               db, tb, cb, disb, hb, zb, ztb):
    wid = _wid()
    r0 = wid * ROWS_W
    pltpu.sync_copy(ck_hbm, cb)
    ck = cb[...][0]
    zero16 = jnp.zeros((16,), f32)
    for q in range(ROWS_W // 16):
        db[pl.ds(q * 16, 16)] = zero16
    for t in range(NW):
        pltpu.sync_copy(degp_hbm.at[pl.ds(t * N_PAD + r0, ROWS_W)], tb)
        for q in range(ROWS_W // 16):
            sl = pl.ds(q * 16, 16)
            db[sl] = db[sl] + tb[sl]
    for q in range(ROWS_W // 16):
        sl = pl.ds(q * 16, 16)
        d = db[sl]
        dv = jnp.where(d > 0.0, _rsqrt_pos(d), jnp.zeros((16,), f32))
        for j in range(16):
            disb[q * 16 + j, :] = jnp.full((16,), dv[j])
    pltpu.sync_copy(disb, disb_hbm.at[pl.ds(r0, ROWS_W)])
    for k in range(NSUB):
        rr = r0 + k * RC
        pltpu.sync_copy(h_hbm.at[pl.ds(rr, RC)], hb)

        @pl.loop(0, RC)
        def _(r):
            d = disb[k * RC + r, :]
            for q in range(C // 16):
                sl = pl.ds(q * 16, 16)
                sh = pl.ds(C + q * 16, 16)
                hrow = hb[r, sl]
                zb[r, sl] = hrow
                zb[r, sh] = ck * hrow
                ztb[r, sl] = d * hrow
                ztb[r, sh] = (ck * d) * hrow

        pltpu.sync_copy(zb, z_hbm.at[pl.ds(rr, RC)])
        pltpu.sync_copy(ztb, zt_hbm.at[pl.ds(rr, RC)])


def _prep(degp, h_pad, ckv):
    sds = jax.ShapeDtypeStruct
    return pl.kernel(
        _prep_body,
        out_type=(sds((N_PAD, 16), f32), sds((N_PAD, CP), f32),
                  sds((N_PAD, CP), f32)),
        mesh=_mesh(),
        compiler_params=_params(),
        scratch_types=[
            pltpu.VMEM((ROWS_W,), f32),
            pltpu.VMEM((ROWS_W,), f32),
            pltpu.VMEM((16,), f32),
            pltpu.VMEM((ROWS_W, 16), f32),
            pltpu.VMEM((RC, C), f32),
            pltpu.VMEM((RC, CP), f32),
            pltpu.VMEM((RC, CP), f32),
        ],
    )(degp, h_pad, ckv)


def _scat_body(vt_hbm, rowr_hbm, colr_hbm, zz_hbm, s_hbm,
               row_v, col_v, rows_b, acc_sh, sem):
    cid = lax.axis_index("c")
    sid = lax.axis_index("s")
    wid = sid * NC + cid

    pltpu.sync_copy(zz_hbm.at[pl.ds(sid * ROWS_SC, ROWS_SC)],
                    acc_sh.at[pl.ds(sid * ROWS_SC, ROWS_SC)])
    plsc.subcore_barrier()

    pltpu.sync_copy(rowr_hbm.at[wid], row_v)
    pltpu.sync_copy(colr_hbm.at[wid], col_v)

    @pl.loop(0, NCHUNK)
    def _(i):
        pltpu.async_copy(vt_hbm.at[row_v.at[i]], rows_b, sem).wait()
        pltpu.sync_copy(rows_b, acc_sh.at[col_v.at[i]], add=True)

    plsc.subcore_barrier()
    pltpu.sync_copy(acc_sh.at[pl.ds(sid * ROWS_SC, ROWS_SC)],
                    s_hbm.at[cid, pl.ds(sid * ROWS_SC, ROWS_SC)])


def _scatter(vt, row_r, col_r, zz):
    return pl.kernel(
        _scat_body,
        out_type=jax.ShapeDtypeStruct((NC, N_PAD, CP), f32),
        mesh=_mesh(),
        compiler_params=_params(),
        scratch_types=[
            pltpu.VMEM((NCHUNK, CHUNK), i32),
            pltpu.VMEM((NCHUNK, CHUNK), i32),
            pltpu.VMEM((CHUNK, CP), f32),
            pltpu.VMEM_SHARED((N_PAD, CP), f32),
            pltpu.SemaphoreType.DMA,
        ],
    )(vt, row_r, col_r, zz)


def _comb_body(z_hbm, s_hbm, disb_hbm, ct_hbm, zo_hbm, zot_hbm,
               vb, s0b, s1b, db, cb, ub, utb):
    wid = _wid()
    pltpu.sync_copy(ct_hbm, cb)
    ct = cb[...][0]
    for k in range(NSUB):
        r0 = wid * ROWS_W + k * RC
        pltpu.sync_copy(z_hbm.at[pl.ds(r0, RC)], vb)
        pltpu.sync_copy(s_hbm.at[0, pl.ds(r0, RC)], s0b)
        pltpu.sync_copy(s_hbm.at[1, pl.ds(r0, RC)], s1b)
        pltpu.sync_copy(disb_hbm.at[pl.ds(r0, RC)], db)

        @pl.loop(0, RC)
        def _(r):
            d = db[r, :]
            mnew = []
            for q in range(C // 16):
                sl = pl.ds(q * 16, 16)
                g = d * (s0b[r, sl] + s1b[r, sl])
                mn = vb[r, sl] + g
                mnew.append(mn)
                ub[r, sl] = mn
                utb[r, sl] = d * mn
            for q in range(C // 16):
                sh = pl.ds(C + q * 16, 16)
                g = d * (s0b[r, sh] + s1b[r, sh])
                an = (vb[r, sh] - g) + ct * mnew[q]
                ub[r, sh] = an
                utb[r, sh] = d * an

        pltpu.sync_copy(ub, zo_hbm.at[pl.ds(r0, RC)])
        pltpu.sync_copy(utb, zot_hbm.at[pl.ds(r0, RC)])


def _combine(z, s, disb, ctv):
    sds = jax.ShapeDtypeStruct
    return pl.kernel(
        _comb_body,
        out_type=(sds((N_PAD, CP), f32), sds((N_PAD, CP), f32)),
        mesh=_mesh(),
        compiler_params=_params(),
        scratch_types=[
            pltpu.VMEM((RC, CP), f32),
            pltpu.VMEM((RC, CP), f32),
            pltpu.VMEM((RC, CP), f32),
            pltpu.VMEM((RC, 16), f32),
            pltpu.VMEM((16,), f32),
            pltpu.VMEM((RC, CP), f32),
            pltpu.VMEM((RC, CP), f32),
        ],
    )(z, s, disb, ctv)


def _mlp_body(x_ref, w1_ref, b1_ref, w2_ref, b2_ref, o_ref):
    h = jnp.dot(x_ref[...], w1_ref[...], preferred_element_type=f32) + b1_ref[...]
    h = jnp.maximum(h, 0.0)
    o_ref[...] = jnp.dot(h, w2_ref[...], preferred_element_type=f32) + b2_ref[...]


def _mlp(x, W1, b1, W2, b2):
    in_c, hid = W1.shape[1], W1.shape[0]
    blk = 1000
    return pl.pallas_call(
        _mlp_body,
        grid=(N // blk,),
        in_specs=[
            pl.BlockSpec((blk, in_c), lambda i: (i, 0)),
            pl.BlockSpec((in_c, hid), lambda i: (0, 0)),
            pl.BlockSpec((1, hid), lambda i: (0, 0)),
            pl.BlockSpec((hid, C), lambda i: (0, 0)),
            pl.BlockSpec((1, C), lambda i: (0, 0)),
        ],
        out_specs=pl.BlockSpec((blk, C), lambda i: (i, 0)),
        out_shape=jax.ShapeDtypeStruct((N, C), f32),
    )(x, W1.T, b1.reshape(1, hid), W2.T, b2.reshape(1, C))


def kernel(x, edge_index, temp, W1, b1, W2, b2):
    row_r = edge_index[0].reshape(NW, NCHUNK, CHUNK)
    col_r = edge_index[1].reshape(NW, NCHUNK, CHUNK)

    h = _mlp(x, W1, b1, W2, b2)
    h_pad = jnp.pad(h, ((0, N_PAD - N), (0, 0)))

    binom = jnp.array([comb(K, i) / 2.0 ** K for i in range(K + 1)], f32)
    coefs = binom * jax.nn.relu(temp)

    degp = _deg_partial(row_r)
    ckv = jnp.zeros((16,), f32).at[0].set(coefs[K])
    disb, z, zt = _prep(degp, h_pad, ckv)

    zz = jnp.zeros((N_PAD, CP), f32)
    for t in range(1, K + 1):
        s = _scatter(zt, row_r, col_r, zz)
        ctv = jnp.zeros((16,), f32).at[0].set(coefs[K - t])
        z, zt = _combine(z, s, disb, ctv)

    return z[:N, C:]


# speedup vs baseline: 59.7791x; 1.5432x over previous
from math import comb

import jax
import jax.numpy as jnp
from jax import lax
from jax.experimental import pallas as pl
from jax.experimental.pallas import tpu as pltpu
from jax.experimental.pallas import tpu_sc as plsc

N = 10000
E = 320000
K = 10
C = 64
CP = 128
NC = 2
NS = 16
NW = NC * NS
N_PAD = 10240
ROWS_W = N_PAD // NW
E_W = E // NW
CHUNK = 80
NCHUNK = E_W // CHUNK
RC = 80
NSUB = ROWS_W // RC
ROWS_SC = N_PAD // NS

f32 = jnp.float32
i32 = jnp.int32


def _mesh():
    return plsc.VectorSubcoreMesh(core_axis_name="c", subcore_axis_name="s")


def _params():
    return pltpu.CompilerParams(needs_layout_passes=False)


def _wid():
    return lax.axis_index("s") * NC + lax.axis_index("c")


def _rsqrt_pos(x):
    yi = jnp.full((16,), 0x5F3759DF, i32) - lax.shift_right_logical(
        plsc.bitcast(x, i32), 1)
    y = plsc.bitcast(yi, f32)
    for _ in range(4):
        y = y * (1.5 - 0.5 * x * y * y)
    return y


def _deg_body(rowr_hbm, degp_hbm, row_v, deg_v):
    wid = _wid()
    pltpu.sync_copy(rowr_hbm.at[wid], row_v)
    zero16 = jnp.zeros((16,), f32)

    @pl.loop(0, N_PAD // 16)
    def _(i):
        deg_v[pl.ds(i * 16, 16)] = zero16

    ones16 = jnp.ones((16,), f32)

    @pl.loop(0, NCHUNK)
    def _(i):
        for k in range(CHUNK // 16):
            idx = row_v[i, pl.ds(k * 16, 16)]
            plsc.addupdate_scatter(deg_v, (idx,), ones16)

    pltpu.sync_copy(deg_v, degp_hbm.at[pl.ds(wid * N_PAD, N_PAD)])


def _deg_partial(row_r):
    return pl.kernel(
        _deg_body,
        out_type=jax.ShapeDtypeStruct((NW * N_PAD,), f32),
        mesh=_mesh(),
        compiler_params=_params(),
        scratch_types=[
            pltpu.VMEM((NCHUNK, CHUNK), i32),
            pltpu.VMEM((N_PAD,), f32),
        ],
    )(row_r)


def _prep_body(degp_hbm, h_hbm, ck_hbm, disb_hbm, z_hbm, zt_hbm,
               db, tb, cb, disb, hb, zb, ztb):
    wid = _wid()
    r0 = wid * ROWS_W
    pltpu.sync_copy(ck_hbm, cb)
    ck = cb[...][0]
    zero16 = jnp.zeros((16,), f32)
    for q in range(ROWS_W // 16):
        db[pl.ds(q * 16, 16)] = zero16
    for t in range(NW):
        pltpu.sync_copy(degp_hbm.at[pl.ds(t * N_PAD + r0, ROWS_W)], tb)
        for q in range(ROWS_W // 16):
            sl = pl.ds(q * 16, 16)
            db[sl] = db[sl] + tb[sl]
    for q in range(ROWS_W // 16):
        sl = pl.ds(q * 16, 16)
        d = db[sl]
        dv = jnp.where(d > 0.0, _rsqrt_pos(d), jnp.zeros((16,), f32))
        for j in range(16):
            disb[q * 16 + j, :] = jnp.full((16,), dv[j])
    pltpu.sync_copy(disb, disb_hbm.at[pl.ds(r0, ROWS_W)])
    for k in range(NSUB):
        rr = r0 + k * RC
        pltpu.sync_copy(h_hbm.at[pl.ds(rr, RC)], hb)

        @pl.loop(0, RC)
        def _(r):
            d = disb[k * RC + r, :]
            for q in range(C // 16):
                sl = pl.ds(q * 16, 16)
                sh = pl.ds(C + q * 16, 16)
                hrow = hb[r, sl]
                zb[r, sl] = hrow
                zb[r, sh] = ck * hrow
                ztb[r, sl] = d * hrow
                ztb[r, sh] = (ck * d) * hrow

        pltpu.sync_copy(zb, z_hbm.at[pl.ds(rr, RC)])
        pltpu.sync_copy(ztb, zt_hbm.at[pl.ds(rr, RC)])


def _prep(degp, h_pad, ckv):
    sds = jax.ShapeDtypeStruct
    return pl.kernel(
        _prep_body,
        out_type=(sds((N_PAD, 16), f32), sds((N_PAD, CP), f32),
                  sds((N_PAD, CP), f32)),
        mesh=_mesh(),
        compiler_params=_params(),
        scratch_types=[
            pltpu.VMEM((ROWS_W,), f32),
            pltpu.VMEM((ROWS_W,), f32),
            pltpu.VMEM((16,), f32),
            pltpu.VMEM((ROWS_W, 16), f32),
            pltpu.VMEM((RC, C), f32),
            pltpu.VMEM((RC, CP), f32),
            pltpu.VMEM((RC, CP), f32),
        ],
    )(degp, h_pad, ckv)


def _scat_body(vt_hbm, row1_hbm, colr_hbm, zz_hbm, s_hbm,
               row_v, col_v, rb2, acc_sh, sem0):
    cid = lax.axis_index("c")
    sid = lax.axis_index("s")
    wid = sid * NC + cid

    pltpu.sync_copy(zz_hbm.at[pl.ds(sid * ROWS_SC, ROWS_SC)],
                    acc_sh.at[pl.ds(sid * ROWS_SC, ROWS_SC)])
    plsc.subcore_barrier()

    pltpu.sync_copy(row1_hbm.at[pl.ds(wid * E_W, E_W)], row_v)
    pltpu.sync_copy(colr_hbm.at[wid], col_v)

    rb0 = rb2.at[pl.ds(0, CHUNK)]
    rb1 = rb2.at[pl.ds(CHUNK, CHUNK)]

    def fire(i, rb):
        pltpu.async_copy(vt_hbm.at[row_v.at[pl.ds(i * CHUNK, CHUNK)]], rb, sem0)

    def drain_scatter(i, rb):
        pltpu.make_async_copy(
            vt_hbm.at[row_v.at[pl.ds(i * CHUNK, CHUNK)]], rb, sem0).wait()
        pltpu.sync_copy(rb, acc_sh.at[col_v.at[i]], add=True)

    fire(0, rb0)

    @pl.loop(0, (NCHUNK - 1) // 2)
    def _(j):
        i = 2 * j
        fire(i + 1, rb1)
        drain_scatter(i, rb0)
        fire(i + 2, rb0)
        drain_scatter(i + 1, rb1)

    drain_scatter(NCHUNK - 1, rb0)

    plsc.subcore_barrier()
    pltpu.sync_copy(acc_sh.at[pl.ds(sid * ROWS_SC, ROWS_SC)],
                    s_hbm.at[cid, pl.ds(sid * ROWS_SC, ROWS_SC)])


def _scatter(vt, row1, col_r, zz):
    return pl.kernel(
        _scat_body,
        out_type=jax.ShapeDtypeStruct((NC, N_PAD, CP), f32),
        mesh=_mesh(),
        compiler_params=_params(),
        scratch_types=[
            pltpu.VMEM((E_W,), i32),
            pltpu.VMEM((NCHUNK, CHUNK), i32),
            pltpu.VMEM((2 * CHUNK, CP), f32),
            pltpu.VMEM_SHARED((N_PAD, CP), f32),
            pltpu.SemaphoreType.DMA,
        ],
    )(vt, row1, col_r, zz)


def _comb_body(z_hbm, s_hbm, disb_hbm, ct_hbm, zo_hbm, zot_hbm,
               vb, s0b, s1b, db, cb, ub, utb):
    wid = _wid()
    pltpu.sync_copy(ct_hbm, cb)
    ct = cb[...][0]
    for k in range(NSUB):
        r0 = wid * ROWS_W + k * RC
        pltpu.sync_copy(z_hbm.at[pl.ds(r0, RC)], vb)
        pltpu.sync_copy(s_hbm.at[0, pl.ds(r0, RC)], s0b)
        pltpu.sync_copy(s_hbm.at[1, pl.ds(r0, RC)], s1b)
        pltpu.sync_copy(disb_hbm.at[pl.ds(r0, RC)], db)

        @pl.loop(0, RC)
        def _(r):
            d = db[r, :]
            mnew = []
            for q in range(C // 16):
                sl = pl.ds(q * 16, 16)
                g = d * (s0b[r, sl] + s1b[r, sl])
                mn = vb[r, sl] + g
                mnew.append(mn)
                ub[r, sl] = mn
                utb[r, sl] = d * mn
            for q in range(C // 16):
                sh = pl.ds(C + q * 16, 16)
                g = d * (s0b[r, sh] + s1b[r, sh])
                an = (vb[r, sh] - g) + ct * mnew[q]
                ub[r, sh] = an
                utb[r, sh] = d * an

        pltpu.sync_copy(ub, zo_hbm.at[pl.ds(r0, RC)])
        pltpu.sync_copy(utb, zot_hbm.at[pl.ds(r0, RC)])


def _combine(z, s, disb, ctv):
    sds = jax.ShapeDtypeStruct
    return pl.kernel(
        _comb_body,
        out_type=(sds((N_PAD, CP), f32), sds((N_PAD, CP), f32)),
        mesh=_mesh(),
        compiler_params=_params(),
        scratch_types=[
            pltpu.VMEM((RC, CP), f32),
            pltpu.VMEM((RC, CP), f32),
            pltpu.VMEM((RC, CP), f32),
            pltpu.VMEM((RC, 16), f32),
            pltpu.VMEM((16,), f32),
            pltpu.VMEM((RC, CP), f32),
            pltpu.VMEM((RC, CP), f32),
        ],
    )(z, s, disb, ctv)


def _mlp_body(x_ref, w1_ref, b1_ref, w2_ref, b2_ref, o_ref):
    h = jnp.dot(x_ref[...], w1_ref[...], preferred_element_type=f32) + b1_ref[...]
    h = jnp.maximum(h, 0.0)
    o_ref[...] = jnp.dot(h, w2_ref[...], preferred_element_type=f32) + b2_ref[...]


def _mlp(x, W1, b1, W2, b2):
    in_c, hid = W1.shape[1], W1.shape[0]
    blk = 1000
    return pl.pallas_call(
        _mlp_body,
        grid=(N // blk,),
        in_specs=[
            pl.BlockSpec((blk, in_c), lambda i: (i, 0)),
            pl.BlockSpec((in_c, hid), lambda i: (0, 0)),
            pl.BlockSpec((1, hid), lambda i: (0, 0)),
            pl.BlockSpec((hid, C), lambda i: (0, 0)),
            pl.BlockSpec((1, C), lambda i: (0, 0)),
        ],
        out_specs=pl.BlockSpec((blk, C), lambda i: (i, 0)),
        out_shape=jax.ShapeDtypeStruct((N, C), f32),
    )(x, W1.T, b1.reshape(1, hid), W2.T, b2.reshape(1, C))


def kernel(x, edge_index, temp, W1, b1, W2, b2):
    row_1 = edge_index[0]
    row_r = edge_index[0].reshape(NW, NCHUNK, CHUNK)
    col_r = edge_index[1].reshape(NW, NCHUNK, CHUNK)

    h = _mlp(x, W1, b1, W2, b2)
    h_pad = jnp.pad(h, ((0, N_PAD - N), (0, 0)))

    binom = jnp.array([comb(K, i) / 2.0 ** K for i in range(K + 1)], f32)
    coefs = binom * jax.nn.relu(temp)

    degp = _deg_partial(row_r)
    ckv = jnp.zeros((16,), f32).at[0].set(coefs[K])
    disb, z, zt = _prep(degp, h_pad, ckv)

    zz = jnp.zeros((N_PAD, CP), f32)
    for t in range(1, K + 1):
        s = _scatter(zt, row_1, col_r, zz)
        ctv = jnp.zeros((16,), f32).at[0].set(coefs[K - t])
        z, zt = _combine(z, s, disb, ctv)

    return z[:N, C:]


# speedup vs baseline: 62.4727x; 1.0451x over previous
from math import comb

import jax
import jax.numpy as jnp
from jax import lax
from jax.experimental import pallas as pl
from jax.experimental.pallas import tpu as pltpu
from jax.experimental.pallas import tpu_sc as plsc

N = 10000
E = 320000
K = 10
C = 64
CP = 128
NC = 2
NS = 16
NW = NC * NS
N_PAD = 10240
ROWS_W = N_PAD // NW
E_W = E // NW
CHUNK = 80
NCHUNK = E_W // CHUNK
RC = 160
NSUB = ROWS_W // RC
ROWS_SC = N_PAD // NS

f32 = jnp.float32
i32 = jnp.int32


def _mesh():
    return plsc.VectorSubcoreMesh(core_axis_name="c", subcore_axis_name="s")


def _params():
    return pltpu.CompilerParams(needs_layout_passes=False)


def _wid():
    return lax.axis_index("s") * NC + lax.axis_index("c")


def _rsqrt_pos(x):
    yi = jnp.full((16,), 0x5F3759DF, i32) - lax.shift_right_logical(
        plsc.bitcast(x, i32), 1)
    y = plsc.bitcast(yi, f32)
    for _ in range(4):
        y = y * (1.5 - 0.5 * x * y * y)
    return y


def _deg_body(rowr_hbm, degp_hbm, row_v, deg_v):
    wid = _wid()
    pltpu.sync_copy(rowr_hbm.at[wid], row_v)
    zero16 = jnp.zeros((16,), f32)

    @pl.loop(0, N_PAD // 16)
    def _(i):
        deg_v[pl.ds(i * 16, 16)] = zero16

    ones16 = jnp.ones((16,), f32)

    @pl.loop(0, NCHUNK)
    def _(i):
        for k in range(CHUNK // 16):
            idx = row_v[i, pl.ds(k * 16, 16)]
            plsc.addupdate_scatter(deg_v, (idx,), ones16)

    pltpu.sync_copy(deg_v, degp_hbm.at[pl.ds(wid * N_PAD, N_PAD)])


def _deg_partial(row_r):
    return pl.kernel(
        _deg_body,
        out_type=jax.ShapeDtypeStruct((NW * N_PAD,), f32),
        mesh=_mesh(),
        compiler_params=_params(),
        scratch_types=[
            pltpu.VMEM((NCHUNK, CHUNK), i32),
            pltpu.VMEM((N_PAD,), f32),
        ],
    )(row_r)


def _prep_body(degp_hbm, h_hbm, ck_hbm, disb_hbm, z_hbm, zt_hbm,
               db, tb, cb, disb, hb, zb, ztb):
    wid = _wid()
    r0 = wid * ROWS_W
    pltpu.sync_copy(ck_hbm, cb)
    ck = cb[...][0]
    zero16 = jnp.zeros((16,), f32)
    for q in range(ROWS_W // 16):
        db[pl.ds(q * 16, 16)] = zero16
    for t in range(NW):
        pltpu.sync_copy(degp_hbm.at[pl.ds(t * N_PAD + r0, ROWS_W)], tb)
        for q in range(ROWS_W // 16):
            sl = pl.ds(q * 16, 16)
            db[sl] = db[sl] + tb[sl]
    for q in range(ROWS_W // 16):
        sl = pl.ds(q * 16, 16)
        d = db[sl]
        dv = jnp.where(d > 0.0, _rsqrt_pos(d), jnp.zeros((16,), f32))
        for j in range(16):
            disb[q * 16 + j, :] = jnp.full((16,), dv[j])
    pltpu.sync_copy(disb, disb_hbm.at[pl.ds(r0, ROWS_W)])
    for k in range(NSUB):
        rr = r0 + k * RC
        pltpu.sync_copy(h_hbm.at[pl.ds(rr, RC)], hb)

        @pl.loop(0, RC)
        def _(r):
            d = disb[k * RC + r, :]
            for q in range(C // 16):
                sl = pl.ds(q * 16, 16)
                sh = pl.ds(C + q * 16, 16)
                hrow = hb[r, sl]
                zb[r, sl] = hrow
                zb[r, sh] = ck * hrow
                ztb[r, sl] = d * hrow
                ztb[r, sh] = (ck * d) * hrow

        pltpu.sync_copy(zb, z_hbm.at[pl.ds(rr, RC)])
        pltpu.sync_copy(ztb, zt_hbm.at[pl.ds(rr, RC)])


def _prep(degp, h_pad, ckv):
    sds = jax.ShapeDtypeStruct
    return pl.kernel(
        _prep_body,
        out_type=(sds((N_PAD, 16), f32), sds((N_PAD, CP), f32),
                  sds((N_PAD, CP), f32)),
        mesh=_mesh(),
        compiler_params=_params(),
        scratch_types=[
            pltpu.VMEM((ROWS_W,), f32),
            pltpu.VMEM((ROWS_W,), f32),
            pltpu.VMEM((16,), f32),
            pltpu.VMEM((ROWS_W, 16), f32),
            pltpu.VMEM((RC, C), f32),
            pltpu.VMEM((RC, CP), f32),
            pltpu.VMEM((RC, CP), f32),
        ],
    )(degp, h_pad, ckv)


def _scat_body(vt_hbm, row1_hbm, colr_hbm, zz_hbm, s_hbm,
               row_v, col_v, rb2, acc_sh, sem0):
    cid = lax.axis_index("c")
    sid = lax.axis_index("s")
    wid = sid * NC + cid

    pltpu.sync_copy(zz_hbm.at[pl.ds(sid * ROWS_SC, ROWS_SC)],
                    acc_sh.at[pl.ds(sid * ROWS_SC, ROWS_SC)])
    plsc.subcore_barrier()

    pltpu.sync_copy(row1_hbm.at[pl.ds(wid * E_W, E_W)], row_v)
    pltpu.sync_copy(colr_hbm.at[wid], col_v)

    rb0 = rb2.at[pl.ds(0, CHUNK)]
    rb1 = rb2.at[pl.ds(CHUNK, CHUNK)]

    def fire(i, rb):
        pltpu.async_copy(vt_hbm.at[row_v.at[pl.ds(i * CHUNK, CHUNK)]], rb, sem0)

    def drain_scatter(i, rb):
        pltpu.make_async_copy(
            vt_hbm.at[row_v.at[pl.ds(i * CHUNK, CHUNK)]], rb, sem0).wait()
        pltpu.sync_copy(rb, acc_sh.at[col_v.at[i]], add=True)

    fire(0, rb0)

    @pl.loop(0, (NCHUNK - 1) // 2)
    def _(j):
        i = 2 * j
        fire(i + 1, rb1)
        drain_scatter(i, rb0)
        fire(i + 2, rb0)
        drain_scatter(i + 1, rb1)

    drain_scatter(NCHUNK - 1, rb0)

    plsc.subcore_barrier()
    pltpu.sync_copy(acc_sh.at[pl.ds(sid * ROWS_SC, ROWS_SC)],
                    s_hbm.at[cid, pl.ds(sid * ROWS_SC, ROWS_SC)])


def _scatter(vt, row1, col_r, zz):
    return pl.kernel(
        _scat_body,
        out_type=jax.ShapeDtypeStruct((NC, N_PAD, CP), f32),
        mesh=_mesh(),
        compiler_params=_params(),
        scratch_types=[
            pltpu.VMEM((E_W,), i32),
            pltpu.VMEM((NCHUNK, CHUNK), i32),
            pltpu.VMEM((2 * CHUNK, CP), f32),
            pltpu.VMEM_SHARED((N_PAD, CP), f32),
            pltpu.SemaphoreType.DMA,
        ],
    )(vt, row1, col_r, zz)


def _comb_body(z_hbm, s_hbm, disb_hbm, ct_hbm, zo_hbm, zot_hbm,
               vb, s0b, s1b, db, cb, ub, utb):
    wid = _wid()
    pltpu.sync_copy(ct_hbm, cb)
    ct = cb[...][0]
    for k in range(NSUB):
        r0 = wid * ROWS_W + k * RC
        pltpu.sync_copy(z_hbm.at[pl.ds(r0, RC)], vb)
        pltpu.sync_copy(s_hbm.at[0, pl.ds(r0, RC)], s0b)
        pltpu.sync_copy(s_hbm.at[1, pl.ds(r0, RC)], s1b)
        pltpu.sync_copy(disb_hbm.at[pl.ds(r0, RC)], db)

        @pl.loop(0, RC)
        def _(r):
            d = db[r, :]
            mnew = []
            for q in range(C // 16):
                sl = pl.ds(q * 16, 16)
                g = d * (s0b[r, sl] + s1b[r, sl])
                mn = vb[r, sl] + g
                mnew.append(mn)
                ub[r, sl] = mn
                utb[r, sl] = d * mn
            for q in range(C // 16):
                sh = pl.ds(C + q * 16, 16)
                g = d * (s0b[r, sh] + s1b[r, sh])
                an = (vb[r, sh] - g) + ct * mnew[q]
                ub[r, sh] = an
                utb[r, sh] = d * an

        pltpu.sync_copy(ub, zo_hbm.at[pl.ds(r0, RC)])
        pltpu.sync_copy(utb, zot_hbm.at[pl.ds(r0, RC)])


def _combine(z, s, disb, ctv):
    sds = jax.ShapeDtypeStruct
    return pl.kernel(
        _comb_body,
        out_type=(sds((N_PAD, CP), f32), sds((N_PAD, CP), f32)),
        mesh=_mesh(),
        compiler_params=_params(),
        scratch_types=[
            pltpu.VMEM((RC, CP), f32),
            pltpu.VMEM((RC, CP), f32),
            pltpu.VMEM((RC, CP), f32),
            pltpu.VMEM((RC, 16), f32),
            pltpu.VMEM((16,), f32),
            pltpu.VMEM((RC, CP), f32),
            pltpu.VMEM((RC, CP), f32),
        ],
    )(z, s, disb, ctv)


def _mlp_body(x_ref, w1_ref, b1_ref, w2_ref, b2_ref, o_ref):
    h = jnp.dot(x_ref[...], w1_ref[...], preferred_element_type=f32) + b1_ref[...]
    h = jnp.maximum(h, 0.0)
    o_ref[...] = jnp.dot(h, w2_ref[...], preferred_element_type=f32) + b2_ref[...]


def _mlp(x, W1, b1, W2, b2):
    in_c, hid = W1.shape[1], W1.shape[0]
    blk = 1000
    return pl.pallas_call(
        _mlp_body,
        grid=(N // blk,),
        in_specs=[
            pl.BlockSpec((blk, in_c), lambda i: (i, 0)),
            pl.BlockSpec((in_c, hid), lambda i: (0, 0)),
            pl.BlockSpec((1, hid), lambda i: (0, 0)),
            pl.BlockSpec((hid, C), lambda i: (0, 0)),
            pl.BlockSpec((1, C), lambda i: (0, 0)),
        ],
        out_specs=pl.BlockSpec((blk, C), lambda i: (i, 0)),
        out_shape=jax.ShapeDtypeStruct((N, C), f32),
    )(x, W1.T, b1.reshape(1, hid), W2.T, b2.reshape(1, C))


def kernel(x, edge_index, temp, W1, b1, W2, b2):
    row_1 = edge_index[0]
    row_r = edge_index[0].reshape(NW, NCHUNK, CHUNK)
    col_r = edge_index[1].reshape(NW, NCHUNK, CHUNK)

    h = _mlp(x, W1, b1, W2, b2)
    h_pad = jnp.pad(h, ((0, N_PAD - N), (0, 0)))

    binom = jnp.array([comb(K, i) / 2.0 ** K for i in range(K + 1)], f32)
    coefs = binom * jax.nn.relu(temp)

    degp = _deg_partial(row_r)
    ckv = jnp.zeros((16,), f32).at[0].set(coefs[K])
    disb, z, zt = _prep(degp, h_pad, ckv)

    zz = jnp.zeros((N_PAD, CP), f32)
    for t in range(1, K + 1):
        s = _scatter(zt, row_1, col_r, zz)
        ctv = jnp.zeros((16,), f32).at[0].set(coefs[K - t])
        z, zt = _combine(z, s, disb, ctv)

    return z[:N, C:]
